# Initial kernel scaffold; baseline (speedup 1.0000x reference)
import jax
import jax.numpy as jnp
from jax.experimental import pallas as pl


def kernel(x, edge_index, batch, params):
    raise NotImplementedError("write your pallas kernel here")


# speedup vs baseline: 5.2875x; 5.2875x over previous
import functools

import jax
import jax.numpy as jnp
from jax import lax
from jax.experimental import pallas as pl
from jax.experimental.pallas import tpu as pltpu
from jax.experimental.pallas import tpu_sc as plsc

N = 50000
E = 800000
D = 64
HD = 32
G = 128

NC, NS = 2, 16
EK = 80
SUB = 5
CHUNK = EK * SUB
EPT = E // NS
OUTER = EPT // CHUNK
ROWS_PT = N // NS

BK = 1000
NBLK = N // BK

_f32 = jnp.float32


def _sc_agg_body(xlo, xhi, src2, dst2, zeros, alo, ahi,
                 acc, sidx, didx, rows, gsem):
    c = lax.axis_index("c")
    s = lax.axis_index("s")

    def run_half(x_hbm, out_hbm):
        pltpu.sync_copy(zeros.at[pl.ds(s * ROWS_PT, ROWS_PT)],
                        acc.at[pl.ds(s * ROWS_PT, ROWS_PT)])
        plsc.subcore_barrier()

        def outer(i, carry):
            row0 = s * (EPT // EK) + i * SUB
            pltpu.sync_copy(src2.at[pl.ds(row0, SUB)], sidx)
            pltpu.sync_copy(dst2.at[pl.ds(row0, SUB)], didx)
            for j in range(SUB):
                pltpu.async_copy(x_hbm.at[sidx.at[j]], rows.at[j], gsem)
            for j in range(SUB):
                pltpu.make_async_copy(x_hbm.at[sidx.at[j]], rows.at[j],
                                      gsem).wait()
            for j in range(SUB):
                pltpu.sync_copy(rows.at[j], acc.at[didx.at[j]], add=True)
            return carry
        lax.fori_loop(0, OUTER, outer, 0)

        plsc.subcore_barrier()
        pltpu.sync_copy(acc.at[pl.ds(s * ROWS_PT, ROWS_PT)],
                        out_hbm.at[pl.ds(s * ROWS_PT, ROWS_PT)])

    @pl.when(c == 0)
    def _():
        run_half(xlo, alo)

    @pl.when(c == 1)
    def _():
        run_half(xhi, ahi)


_sc_agg = pl.kernel(
    _sc_agg_body,
    out_type=(jax.ShapeDtypeStruct((N, HD), _f32),
              jax.ShapeDtypeStruct((N, HD), _f32)),
    mesh=plsc.VectorSubcoreMesh(core_axis_name="c", subcore_axis_name="s",
                                num_cores=NC, num_subcores=NS),
    scratch_types=[
        pltpu.VMEM_SHARED((N, HD), _f32),
        pltpu.VMEM((SUB, EK), jnp.int32),
        pltpu.VMEM((SUB, EK), jnp.int32),
        pltpu.VMEM((SUB, EK, HD), _f32),
        pltpu.SemaphoreType.DMA,
    ],
    compiler_params=pltpu.CompilerParams(use_tc_tiling_on_sc=False),
)


def _pass1_body(xlo, xhi, alo, ahi, w1, b1, y, ssum, ssq):
    i = pl.program_id(0)
    h = jnp.concatenate([xlo[...] + alo[...], xhi[...] + ahi[...]], axis=1)
    yv = jnp.dot(h, w1[...], preferred_element_type=_f32) + b1[...]
    y[...] = yv

    @pl.when(i == 0)
    def _():
        ssum[...] = jnp.zeros_like(ssum)
        ssq[...] = jnp.zeros_like(ssq)

    ssum[...] += jnp.sum(yv, axis=0, keepdims=True)
    ssq[...] += jnp.sum(yv * yv, axis=0, keepdims=True)


_tc_pass1 = pl.pallas_call(
    _pass1_body,
    grid=(NBLK,),
    in_specs=[
        pl.BlockSpec((BK, HD), lambda i: (i, 0)),
        pl.BlockSpec((BK, HD), lambda i: (i, 0)),
        pl.BlockSpec((BK, HD), lambda i: (i, 0)),
        pl.BlockSpec((BK, HD), lambda i: (i, 0)),
        pl.BlockSpec((D, D), lambda i: (0, 0)),
        pl.BlockSpec((1, D), lambda i: (0, 0)),
    ],
    out_specs=[
        pl.BlockSpec((BK, D), lambda i: (i, 0)),
        pl.BlockSpec((1, D), lambda i: (0, 0)),
        pl.BlockSpec((1, D), lambda i: (0, 0)),
    ],
    out_shape=[
        jax.ShapeDtypeStruct((N, D), _f32),
        jax.ShapeDtypeStruct((1, D), _f32),
        jax.ShapeDtypeStruct((1, D), _f32),
    ],
)


def _pass2_body(y, ssum, ssq, gamma, beta, w2, b2, hlo, hhi):
    mean = ssum[...] * (1.0 / N)
    var = ssq[...] * (1.0 / N) - mean * mean
    inv = lax.rsqrt(var + 1e-5) * gamma[...]
    z = jnp.maximum((y[...] - mean) * inv + beta[...], 0.0)
    hv = jnp.dot(z, w2[...], preferred_element_type=_f32) + b2[...]
    hv = jnp.maximum(hv, 0.0)
    hlo[...] = hv[:, :HD]
    hhi[...] = hv[:, HD:]


_tc_pass2 = pl.pallas_call(
    _pass2_body,
    grid=(NBLK,),
    in_specs=[
        pl.BlockSpec((BK, D), lambda i: (i, 0)),
        pl.BlockSpec((1, D), lambda i: (0, 0)),
        pl.BlockSpec((1, D), lambda i: (0, 0)),
        pl.BlockSpec((1, D), lambda i: (0, 0)),
        pl.BlockSpec((1, D), lambda i: (0, 0)),
        pl.BlockSpec((D, D), lambda i: (0, 0)),
        pl.BlockSpec((1, D), lambda i: (0, 0)),
    ],
    out_specs=[
        pl.BlockSpec((BK, HD), lambda i: (i, 0)),
        pl.BlockSpec((BK, HD), lambda i: (i, 0)),
    ],
    out_shape=[
        jax.ShapeDtypeStruct((N, HD), _f32),
        jax.ShapeDtypeStruct((N, HD), _f32),
    ],
)


def _pool_body(hlo, hhi, batch3, batchc, wc1, bc1, wc2, bc2, out,
               psum, pcnt, pmax):
    i = pl.program_id(0)

    @pl.when(i == 0)
    def _():
        psum[...] = jnp.zeros_like(psum)
        pcnt[...] = jnp.zeros_like(pcnt)
        pmax[...] = jnp.full_like(pmax, -jnp.inf)

    brow = batch3[0]
    bcol = batchc[...]
    h = jnp.concatenate([hlo[...], hhi[...]], axis=1)
    gids = lax.broadcasted_iota(jnp.int32, (G, BK), 0)
    oh = (gids == brow).astype(_f32)
    psum[...] += jnp.dot(oh, h, preferred_element_type=_f32)
    pcnt[...] += jnp.dot(oh, jnp.ones((BK, 8), _f32),
                         preferred_element_type=_f32)

    gmin = batch3[0, 0, 0]
    gmax = batch3[0, 0, BK - 1]

    def mb(g, carry):
        @pl.when((g >= gmin) & (g <= gmax))
        def _():
            m = jnp.where(bcol == g, h, -jnp.inf)
            cm = jnp.max(m, axis=0, keepdims=True)
            pmax[pl.ds(g, 1), :] = jnp.maximum(pmax[pl.ds(g, 1), :], cm)
        return carry
    lax.fori_loop(0, G, mb, 0)

    @pl.when(i == NBLK - 1)
    def _():
        mean_pool = psum[...] / jnp.maximum(pcnt[:, 0:1], 1.0)
        gg = jnp.concatenate([mean_pool, pmax[...]], axis=1)
        t = jnp.maximum(
            jnp.dot(gg, wc1[...], preferred_element_type=_f32) + bc1[...], 0.0)
        out[...] = jnp.dot(t, wc2[...], preferred_element_type=_f32) + bc2[...]


_tc_pool = pl.pallas_call(
    _pool_body,
    grid=(NBLK,),
    in_specs=[
        pl.BlockSpec((BK, HD), lambda i: (i, 0)),
        pl.BlockSpec((BK, HD), lambda i: (i, 0)),
        pl.BlockSpec((1, 1, BK), lambda i: (i, 0, 0)),
        pl.BlockSpec((BK, 1), lambda i: (i, 0)),
        pl.BlockSpec((2 * D, D), lambda i: (0, 0)),
        pl.BlockSpec((1, D), lambda i: (0, 0)),
        pl.BlockSpec((D, 128), lambda i: (0, 0)),
        pl.BlockSpec((1, 128), lambda i: (0, 0)),
    ],
    out_specs=pl.BlockSpec((G, 128), lambda i: (0, 0)),
    out_shape=jax.ShapeDtypeStruct((G, 128), _f32),
    scratch_shapes=[
        pltpu.VMEM((G, D), _f32),
        pltpu.VMEM((G, 8), _f32),
        pltpu.VMEM((G, D), _f32),
    ],
)


def kernel(x, edge_index, batch, params):
    src2 = edge_index[0].reshape(E // EK, EK)
    dst2 = edge_index[1].reshape(E // EK, EK)
    zeros = jnp.zeros((N, HD), _f32)
    h_lo = x[:, :HD]
    h_hi = x[:, HD:]
    for name in ("conv1", "conv2", "conv3"):
        p = params[name]
        a_lo, a_hi = _sc_agg(h_lo, h_hi, src2, dst2, zeros)
        y, ssum, ssq = _tc_pass1(h_lo, h_hi, a_lo, a_hi,
                                 p["W1"], p["b1"].reshape(1, D))
        h_lo, h_hi = _tc_pass2(y, ssum, ssq,
                               p["gamma"].reshape(1, D),
                               p["beta"].reshape(1, D),
                               p["W2"], p["b2"].reshape(1, D))
    c = params["cls"]
    batch3 = batch.reshape(NBLK, 1, BK)
    batchc = batch.reshape(N, 1)
    w2p = jnp.pad(c["W2"], ((0, 0), (0, 128 - 2)))
    b2p = jnp.pad(c["b2"].reshape(1, 2), ((0, 0), (0, 128 - 2)))
    outp = _tc_pool(h_lo, h_hi, batch3, batchc, c["W1"],
                    c["b1"].reshape(1, D), w2p, b2p)
    return outp[:, :2]


# speedup vs baseline: 6.6701x; 1.2615x over previous
import functools

import jax
import jax.numpy as jnp
from jax import lax
from jax.experimental import pallas as pl
from jax.experimental.pallas import tpu as pltpu
from jax.experimental.pallas import tpu_sc as plsc

N = 50000
E = 800000
D = 64
HD = 32
G = 128

NC, NS = 2, 16
EK = 80
SUB = 5
CHUNK = EK * SUB
EPT = E // NS
OUTER = EPT // CHUNK
ROWS_PT = N // NS

BK = 1000
NBLK = N // BK

_f32 = jnp.float32


def _sc_agg_body(xlo, xhi, src2, dst2, zeros, alo, ahi,
                 acc, sidx0, didx0, rows0, sidx1, didx1, rows1, gsem0, gsem1):
    c = lax.axis_index("c")
    s = lax.axis_index("s")

    def run_half(x_hbm, out_hbm):
        pltpu.sync_copy(zeros.at[pl.ds(s * ROWS_PT, ROWS_PT)],
                        acc.at[pl.ds(s * ROWS_PT, ROWS_PT)])
        plsc.subcore_barrier()

        def fire(i, sx, dx, rw, sem):
            row0 = s * (EPT // EK) + i * SUB
            pltpu.sync_copy(src2.at[pl.ds(row0, SUB)], sx)
            pltpu.sync_copy(dst2.at[pl.ds(row0, SUB)], dx)
            for j in range(SUB):
                pltpu.async_copy(x_hbm.at[sx.at[j]], rw.at[j], sem)

        def drain_scatter(sx, dx, rw, sem):
            for j in range(SUB):
                pltpu.make_async_copy(x_hbm.at[sx.at[j]], rw.at[j],
                                      sem).wait()
            for j in range(SUB):
                pltpu.sync_copy(rw.at[j], acc.at[dx.at[j]], add=True)

        fire(0, sidx0, didx0, rows0, gsem0)

        def outer(k, carry):
            i = 2 * k
            fire(i + 1, sidx1, didx1, rows1, gsem1)
            drain_scatter(sidx0, didx0, rows0, gsem0)
            fire(i + 2, sidx0, didx0, rows0, gsem0)
            drain_scatter(sidx1, didx1, rows1, gsem1)
            return carry
        lax.fori_loop(0, (OUTER - 1) // 2, outer, 0)
        drain_scatter(sidx0, didx0, rows0, gsem0)

        plsc.subcore_barrier()
        pltpu.sync_copy(acc.at[pl.ds(s * ROWS_PT, ROWS_PT)],
                        out_hbm.at[pl.ds(s * ROWS_PT, ROWS_PT)])

    @pl.when(c == 0)
    def _():
        run_half(xlo, alo)

    @pl.when(c == 1)
    def _():
        run_half(xhi, ahi)


_sc_agg = pl.kernel(
    _sc_agg_body,
    out_type=(jax.ShapeDtypeStruct((N, HD), _f32),
              jax.ShapeDtypeStruct((N, HD), _f32)),
    mesh=plsc.VectorSubcoreMesh(core_axis_name="c", subcore_axis_name="s",
                                num_cores=NC, num_subcores=NS),
    scratch_types=[
        pltpu.VMEM_SHARED((N, HD), _f32),
        pltpu.VMEM((SUB, EK), jnp.int32),
        pltpu.VMEM((SUB, EK), jnp.int32),
        pltpu.VMEM((SUB, EK, HD), _f32),
        pltpu.VMEM((SUB, EK), jnp.int32),
        pltpu.VMEM((SUB, EK), jnp.int32),
        pltpu.VMEM((SUB, EK, HD), _f32),
        pltpu.SemaphoreType.DMA,
        pltpu.SemaphoreType.DMA,
    ],
    compiler_params=pltpu.CompilerParams(use_tc_tiling_on_sc=False),
)


def _pass1_body(xlo, xhi, alo, ahi, w1, b1, y, ssum, ssq):
    i = pl.program_id(0)
    h = jnp.concatenate([xlo[...] + alo[...], xhi[...] + ahi[...]], axis=1)
    yv = jnp.dot(h, w1[...], preferred_element_type=_f32) + b1[...]
    y[...] = yv

    @pl.when(i == 0)
    def _():
        ssum[...] = jnp.zeros_like(ssum)
        ssq[...] = jnp.zeros_like(ssq)

    ssum[...] += jnp.sum(yv, axis=0, keepdims=True)
    ssq[...] += jnp.sum(yv * yv, axis=0, keepdims=True)


_tc_pass1 = pl.pallas_call(
    _pass1_body,
    grid=(NBLK,),
    in_specs=[
        pl.BlockSpec((BK, HD), lambda i: (i, 0)),
        pl.BlockSpec((BK, HD), lambda i: (i, 0)),
        pl.BlockSpec((BK, HD), lambda i: (i, 0)),
        pl.BlockSpec((BK, HD), lambda i: (i, 0)),
        pl.BlockSpec((D, D), lambda i: (0, 0)),
        pl.BlockSpec((1, D), lambda i: (0, 0)),
    ],
    out_specs=[
        pl.BlockSpec((BK, D), lambda i: (i, 0)),
        pl.BlockSpec((1, D), lambda i: (0, 0)),
        pl.BlockSpec((1, D), lambda i: (0, 0)),
    ],
    out_shape=[
        jax.ShapeDtypeStruct((N, D), _f32),
        jax.ShapeDtypeStruct((1, D), _f32),
        jax.ShapeDtypeStruct((1, D), _f32),
    ],
)


def _pass2_body(y, ssum, ssq, gamma, beta, w2, b2, hlo, hhi):
    mean = ssum[...] * (1.0 / N)
    var = ssq[...] * (1.0 / N) - mean * mean
    inv = lax.rsqrt(var + 1e-5) * gamma[...]
    z = jnp.maximum((y[...] - mean) * inv + beta[...], 0.0)
    hv = jnp.dot(z, w2[...], preferred_element_type=_f32) + b2[...]
    hv = jnp.maximum(hv, 0.0)
    hlo[...] = hv[:, :HD]
    hhi[...] = hv[:, HD:]


_tc_pass2 = pl.pallas_call(
    _pass2_body,
    grid=(NBLK,),
    in_specs=[
        pl.BlockSpec((BK, D), lambda i: (i, 0)),
        pl.BlockSpec((1, D), lambda i: (0, 0)),
        pl.BlockSpec((1, D), lambda i: (0, 0)),
        pl.BlockSpec((1, D), lambda i: (0, 0)),
        pl.BlockSpec((1, D), lambda i: (0, 0)),
        pl.BlockSpec((D, D), lambda i: (0, 0)),
        pl.BlockSpec((1, D), lambda i: (0, 0)),
    ],
    out_specs=[
        pl.BlockSpec((BK, HD), lambda i: (i, 0)),
        pl.BlockSpec((BK, HD), lambda i: (i, 0)),
    ],
    out_shape=[
        jax.ShapeDtypeStruct((N, HD), _f32),
        jax.ShapeDtypeStruct((N, HD), _f32),
    ],
)


def _pool_body(hlo, hhi, batch3, batchc, wc1, bc1, wc2, bc2, out,
               psum, pcnt, pmax):
    i = pl.program_id(0)

    @pl.when(i == 0)
    def _():
        psum[...] = jnp.zeros_like(psum)
        pcnt[...] = jnp.zeros_like(pcnt)
        pmax[...] = jnp.full_like(pmax, -jnp.inf)

    brow = batch3[0]
    bcol = batchc[...]
    h = jnp.concatenate([hlo[...], hhi[...]], axis=1)
    gids = lax.broadcasted_iota(jnp.int32, (G, BK), 0)
    oh = (gids == brow).astype(_f32)
    psum[...] += jnp.dot(oh, h, preferred_element_type=_f32)
    pcnt[...] += jnp.dot(oh, jnp.ones((BK, 8), _f32),
                         preferred_element_type=_f32)

    gmin = batch3[0, 0, 0]
    gmax = batch3[0, 0, BK - 1]

    def mb(g, carry):
        @pl.when((g >= gmin) & (g <= gmax))
        def _():
            m = jnp.where(bcol == g, h, -jnp.inf)
            cm = jnp.max(m, axis=0, keepdims=True)
            pmax[pl.ds(g, 1), :] = jnp.maximum(pmax[pl.ds(g, 1), :], cm)
        return carry
    lax.fori_loop(0, G, mb, 0)

    @pl.when(i == NBLK - 1)
    def _():
        mean_pool = psum[...] / jnp.maximum(pcnt[:, 0:1], 1.0)
        gg = jnp.concatenate([mean_pool, pmax[...]], axis=1)
        t = jnp.maximum(
            jnp.dot(gg, wc1[...], preferred_element_type=_f32) + bc1[...], 0.0)
        out[...] = jnp.dot(t, wc2[...], preferred_element_type=_f32) + bc2[...]


_tc_pool = pl.pallas_call(
    _pool_body,
    grid=(NBLK,),
    in_specs=[
        pl.BlockSpec((BK, HD), lambda i: (i, 0)),
        pl.BlockSpec((BK, HD), lambda i: (i, 0)),
        pl.BlockSpec((1, 1, BK), lambda i: (i, 0, 0)),
        pl.BlockSpec((BK, 1), lambda i: (i, 0)),
        pl.BlockSpec((2 * D, D), lambda i: (0, 0)),
        pl.BlockSpec((1, D), lambda i: (0, 0)),
        pl.BlockSpec((D, 128), lambda i: (0, 0)),
        pl.BlockSpec((1, 128), lambda i: (0, 0)),
    ],
    out_specs=pl.BlockSpec((G, 128), lambda i: (0, 0)),
    out_shape=jax.ShapeDtypeStruct((G, 128), _f32),
    scratch_shapes=[
        pltpu.VMEM((G, D), _f32),
        pltpu.VMEM((G, 8), _f32),
        pltpu.VMEM((G, D), _f32),
    ],
)


def kernel(x, edge_index, batch, params):
    src2 = edge_index[0].reshape(E // EK, EK)
    dst2 = edge_index[1].reshape(E // EK, EK)
    zeros = jnp.zeros((N, HD), _f32)
    h_lo = x[:, :HD]
    h_hi = x[:, HD:]
    for name in ("conv1", "conv2", "conv3"):
        p = params[name]
        a_lo, a_hi = _sc_agg(h_lo, h_hi, src2, dst2, zeros)
        y, ssum, ssq = _tc_pass1(h_lo, h_hi, a_lo, a_hi,
                                 p["W1"], p["b1"].reshape(1, D))
        h_lo, h_hi = _tc_pass2(y, ssum, ssq,
                               p["gamma"].reshape(1, D),
                               p["beta"].reshape(1, D),
                               p["W2"], p["b2"].reshape(1, D))
    c = params["cls"]
    batch3 = batch.reshape(NBLK, 1, BK)
    batchc = batch.reshape(N, 1)
    w2p = jnp.pad(c["W2"], ((0, 0), (0, 128 - 2)))
    b2p = jnp.pad(c["b2"].reshape(1, 2), ((0, 0), (0, 128 - 2)))
    outp = _tc_pool(h_lo, h_hi, batch3, batchc, c["W1"],
                    c["b1"].reshape(1, D), w2p, b2p)
    return outp[:, :2]
